# Initial kernel scaffold; baseline (speedup 1.0000x reference)
import jax
import jax.numpy as jnp
from jax.experimental import pallas as pl


def kernel(data1, data2, segment_ids):
    raise NotImplementedError("write your pallas kernel here")


# speedup vs baseline: 3.1143x; 3.1143x over previous
import functools

import jax
import jax.numpy as jnp
from jax import lax
from jax.experimental import pallas as pl
from jax.experimental.pallas import tpu as pltpu
from jax.experimental.pallas import tpu_sc as plsc

N = 16384
D = 256
M = 512
S = 1024

NUM_CORES = 2
NUM_SUBCORES = 16
NUM_WORKERS = NUM_CORES * NUM_SUBCORES
SEGS_PER_WORKER = S // NUM_WORKERS
CHUNK = 128
LANES = 16
D_VECS = D // LANES


def _lower_bound_vec(seg_v, targets):
    pos = jnp.zeros((LANES,), jnp.int32)
    sz = N // 2
    while sz >= 1:
        probe = plsc.load_gather(seg_v, [pos + (sz - 1)])
        pos = jnp.where(probe < targets, pos + sz, pos)
        sz //= 2
    probe = plsc.load_gather(seg_v, [pos])
    return jnp.where(probe < targets, pos + 1, pos)


def _sc_body(data1_hbm, seg_hbm, z_acc_hbm, agg_out, cnt_out,
             seg_v, buf_v, acc_v, cnt_v):
    c = lax.axis_index("c")
    s = lax.axis_index("s")
    wid = c * NUM_SUBCORES + s
    s_lo = wid * SEGS_PER_WORKER

    pltpu.sync_copy(seg_hbm, seg_v.at[pl.ds(0, N)])
    pltpu.sync_copy(z_acc_hbm, acc_v)

    lane = lax.iota(jnp.int32, LANES)
    lb0 = _lower_bound_vec(seg_v, s_lo + lane)
    lb1 = _lower_bound_vec(seg_v, s_lo + LANES + lane)
    ub0 = _lower_bound_vec(seg_v, s_lo + 1 + lane)
    ub1 = _lower_bound_vec(seg_v, s_lo + LANES + 1 + lane)
    cnt_v[pl.ds(0, LANES)] = (ub0 - lb0).astype(jnp.float32)
    cnt_v[pl.ds(LANES, LANES)] = (ub1 - lb1).astype(jnp.float32)
    pltpu.sync_copy(cnt_v, cnt_out.at[pl.ds(s_lo, SEGS_PER_WORKER)])

    r_lo = lb0[0]
    r_hi = ub1[LANES - 1]
    c_lo = lax.shift_right_logical(r_lo, 7)
    c_hi = lax.shift_right_logical(r_hi + (CHUNK - 1), 7)

    def chunk_body(ci, _):
        row0 = ci * CHUNK
        pltpu.sync_copy(data1_hbm.at[pl.ds(row0, CHUNK)], buf_v)

        def row_body(r, _):
            j = seg_v[pl.ds(r, LANES)][0] - s_lo
            rr = r - row0
            for k in range(D_VECS):
                plsc.addupdate(acc_v.at[j, pl.ds(k * LANES, LANES)],
                               buf_v[rr, pl.ds(k * LANES, LANES)])
            return 0

        lax.fori_loop(jnp.maximum(r_lo, row0),
                      jnp.minimum(r_hi, row0 + CHUNK),
                      row_body, 0, unroll=False)
        return 0

    lax.fori_loop(c_lo, c_hi, chunk_body, 0, unroll=False)

    pltpu.sync_copy(acc_v, agg_out.at[pl.ds(s_lo, SEGS_PER_WORKER)])


@functools.cache
def _sc_segment_sum():
    return pl.kernel(
        _sc_body,
        out_type=(
            jax.ShapeDtypeStruct((S, D), jnp.float32),
            jax.ShapeDtypeStruct((S,), jnp.float32),
        ),
        mesh=plsc.VectorSubcoreMesh(core_axis_name="c", subcore_axis_name="s",
                                    num_cores=NUM_CORES,
                                    num_subcores=NUM_SUBCORES),
        compiler_params=pltpu.CompilerParams(needs_layout_passes=False),
        scratch_types=[
            pltpu.VMEM((N + LANES,), jnp.int32),
            pltpu.VMEM((CHUNK, D), jnp.float32),
            pltpu.VMEM((SEGS_PER_WORKER, D), jnp.float32),
            pltpu.VMEM((SEGS_PER_WORKER,), jnp.float32),
        ],
    )


def _tc_body(agg_ref, cnt_ref, d2_ref, out_ref):
    mat = lax.dot_general(agg_ref[...], d2_ref[...],
                          dimension_numbers=(((1,), (1,)), ((), ())),
                          preferred_element_type=jnp.float32)
    out_ref[...] = mat / jnp.maximum(cnt_ref[...], 1.0)


_tc_matmul = pl.pallas_call(
    _tc_body,
    out_shape=jax.ShapeDtypeStruct((S, M), jnp.float32),
)


@jax.jit
def kernel(data1, data2, segment_ids):
    seg = segment_ids.astype(jnp.int32)
    z_acc = jnp.zeros((SEGS_PER_WORKER, D), jnp.float32)
    agg, cnt = _sc_segment_sum()(data1, seg, z_acc)
    return _tc_matmul(agg, cnt.reshape(S, 1), data2)


# speedup vs baseline: 4.4091x; 1.4157x over previous
import functools

import jax
import jax.numpy as jnp
from jax import lax
from jax.experimental import pallas as pl
from jax.experimental.pallas import tpu as pltpu
from jax.experimental.pallas import tpu_sc as plsc

N = 16384
D = 256
M = 512
S = 1024

NUM_CORES = 2
NUM_SUBCORES = 16
NUM_WORKERS = NUM_CORES * NUM_SUBCORES
SEGS_PER_WORKER = S // NUM_WORKERS
CHUNK = 128
LANES = 16
D_VECS = D // LANES


def _lower_bound_vec(seg_v, targets):
    pos = jnp.zeros((LANES,), jnp.int32)
    sz = N // 2
    while sz >= 1:
        probe = plsc.load_gather(seg_v, [pos + (sz - 1)])
        pos = jnp.where(probe < targets, pos + sz, pos)
        sz //= 2
    probe = plsc.load_gather(seg_v, [pos])
    return jnp.where(probe < targets, pos + 1, pos)


def _sc_body(data1_hbm, seg_hbm, z_acc_hbm, agg_out, cnt_out,
             seg_v, buf_v, acc_v, cnt_v):
    c = lax.axis_index("c")
    s = lax.axis_index("s")
    wid = c * NUM_SUBCORES + s
    s_lo = wid * SEGS_PER_WORKER

    pltpu.sync_copy(seg_hbm, seg_v.at[pl.ds(0, N)])
    pltpu.sync_copy(z_acc_hbm, acc_v)

    lane = lax.iota(jnp.int32, LANES)
    lb0 = _lower_bound_vec(seg_v, s_lo + lane)
    lb1 = _lower_bound_vec(seg_v, s_lo + LANES + lane)
    ub0 = _lower_bound_vec(seg_v, s_lo + 1 + lane)
    ub1 = _lower_bound_vec(seg_v, s_lo + LANES + 1 + lane)
    cnt_v[pl.ds(0, LANES)] = (ub0 - lb0).astype(jnp.float32)
    cnt_v[pl.ds(LANES, LANES)] = (ub1 - lb1).astype(jnp.float32)
    pltpu.sync_copy(cnt_v, cnt_out.at[pl.ds(s_lo, SEGS_PER_WORKER)])

    r_lo = lb0[0]
    r_hi = ub1[LANES - 1]
    c_lo = lax.shift_right_logical(r_lo, 7)
    c_hi = lax.shift_right_logical(r_hi + (CHUNK - 1), 7)

    zero = jnp.zeros((LANES,), jnp.float32)
    init = (jnp.int32(-1),) + (zero,) * D_VECS

    def chunk_body(ci, carry):
        row0 = ci * CHUNK
        pltpu.sync_copy(data1_hbm.at[pl.ds(row0, CHUNK)], buf_v)

        def row_body(r, carry):
            prev_j, *accs = carry
            j = seg_v[pl.ds(r, LANES)][0] - s_lo
            flush = j != prev_j

            @pl.when(flush & (prev_j >= 0))
            def _():
                for k in range(D_VECS):
                    acc_v[prev_j, pl.ds(k * LANES, LANES)] = accs[k]

            rr = r - row0
            new = [
                jnp.where(flush, row_k, acc_k + row_k)
                for k in range(D_VECS)
                for row_k, acc_k in ((buf_v[rr, pl.ds(k * LANES, LANES)],
                                      accs[k]),)
            ]
            return (j, *new)

        return lax.fori_loop(jnp.maximum(r_lo, row0),
                             jnp.minimum(r_hi, row0 + CHUNK),
                             row_body, carry, unroll=False)

    final = lax.fori_loop(c_lo, c_hi, chunk_body, init, unroll=False)
    last_j, *last_accs = final

    @pl.when(last_j >= 0)
    def _():
        for k in range(D_VECS):
            acc_v[last_j, pl.ds(k * LANES, LANES)] = last_accs[k]

    pltpu.sync_copy(acc_v, agg_out.at[pl.ds(s_lo, SEGS_PER_WORKER)])


@functools.cache
def _sc_segment_sum():
    return pl.kernel(
        _sc_body,
        out_type=(
            jax.ShapeDtypeStruct((S, D), jnp.float32),
            jax.ShapeDtypeStruct((S,), jnp.float32),
        ),
        mesh=plsc.VectorSubcoreMesh(core_axis_name="c", subcore_axis_name="s",
                                    num_cores=NUM_CORES,
                                    num_subcores=NUM_SUBCORES),
        compiler_params=pltpu.CompilerParams(needs_layout_passes=False),
        scratch_types=[
            pltpu.VMEM((N + LANES,), jnp.int32),
            pltpu.VMEM((CHUNK, D), jnp.float32),
            pltpu.VMEM((SEGS_PER_WORKER, D), jnp.float32),
            pltpu.VMEM((SEGS_PER_WORKER,), jnp.float32),
        ],
    )


def _tc_body(agg_ref, cnt_ref, d2_ref, out_ref):
    mat = lax.dot_general(agg_ref[...], d2_ref[...],
                          dimension_numbers=(((1,), (1,)), ((), ())),
                          preferred_element_type=jnp.float32)
    out_ref[...] = mat / jnp.maximum(cnt_ref[...], 1.0)


_tc_matmul = pl.pallas_call(
    _tc_body,
    out_shape=jax.ShapeDtypeStruct((S, M), jnp.float32),
)


@jax.jit
def kernel(data1, data2, segment_ids):
    seg = segment_ids.astype(jnp.int32)
    z_acc = jnp.zeros((SEGS_PER_WORKER, D), jnp.float32)
    agg, cnt = _sc_segment_sum()(data1, seg, z_acc)
    return _tc_matmul(agg, cnt.reshape(S, 1), data2)


# speedup vs baseline: 4.9997x; 1.1340x over previous
import functools

import jax
import jax.numpy as jnp
from jax import lax
from jax.experimental import pallas as pl
from jax.experimental.pallas import tpu as pltpu
from jax.experimental.pallas import tpu_sc as plsc

N = 16384
D = 256
M = 512
S = 1024

NUM_CORES = 2
NUM_SUBCORES = 16
NUM_WORKERS = NUM_CORES * NUM_SUBCORES
SEGS_PER_WORKER = S // NUM_WORKERS
CHUNK = 128
LANES = 16
D_VECS = D // LANES


def _lower_bound_vec(seg_v, targets):
    pos = jnp.zeros((LANES,), jnp.int32)
    sz = N // 2
    while sz >= 1:
        probe = plsc.load_gather(seg_v, [pos + (sz - 1)])
        pos = jnp.where(probe < targets, pos + sz, pos)
        sz //= 2
    probe = plsc.load_gather(seg_v, [pos])
    return jnp.where(probe < targets, pos + 1, pos)


def _sc_body(data1_hbm, seg_hbm, z_acc_hbm, agg_out, cnt_out,
             seg_v, buf_v, buf2_v, acc_v, cnt_v, sem0, sem1):
    c = lax.axis_index("c")
    s = lax.axis_index("s")
    wid = c * NUM_SUBCORES + s
    s_lo = wid * SEGS_PER_WORKER

    pltpu.sync_copy(seg_hbm, seg_v.at[pl.ds(0, N)])
    pltpu.sync_copy(z_acc_hbm, acc_v)

    lane = lax.iota(jnp.int32, LANES)
    lb0 = _lower_bound_vec(seg_v, s_lo + lane)
    lb1 = _lower_bound_vec(seg_v, s_lo + LANES + lane)
    ub0 = _lower_bound_vec(seg_v, s_lo + 1 + lane)
    ub1 = _lower_bound_vec(seg_v, s_lo + LANES + 1 + lane)
    cnt_v[pl.ds(0, LANES)] = (ub0 - lb0).astype(jnp.float32)
    cnt_v[pl.ds(LANES, LANES)] = (ub1 - lb1).astype(jnp.float32)
    pltpu.sync_copy(cnt_v, cnt_out.at[pl.ds(s_lo, SEGS_PER_WORKER)])

    r_lo = lb0[0]
    r_hi = ub1[LANES - 1]
    c_lo = lax.shift_right_logical(r_lo, 7)
    c_hi = lax.shift_right_logical(r_hi + (CHUNK - 1), 7)

    zero = jnp.zeros((LANES,), jnp.float32)
    init = (jnp.int32(-1),) + (zero,) * D_VECS

    def process_chunk(ci, buf, carry):
        row0 = ci * CHUNK

        def row_body(r, carry):
            prev_j, *accs = carry
            j = seg_v[pl.ds(r, LANES)][0] - s_lo
            flush = j != prev_j

            @pl.when(flush & (prev_j >= 0))
            def _():
                for k in range(D_VECS):
                    acc_v[prev_j, pl.ds(k * LANES, LANES)] = accs[k]

            rr = r - row0
            new = [
                jnp.where(flush, row_k, acc_k + row_k)
                for k in range(D_VECS)
                for row_k, acc_k in ((buf[rr, pl.ds(k * LANES, LANES)],
                                      accs[k]),)
            ]
            return (j, *new)

        return lax.fori_loop(jnp.maximum(r_lo, row0),
                             jnp.minimum(r_hi, row0 + CHUNK),
                             row_body, carry, unroll=False)

    def issue(ci, buf, sem):
        pltpu.async_copy(data1_hbm.at[pl.ds(ci * CHUNK, CHUNK)], buf, sem)

    def drain(buf, sem):
        pltpu.make_async_copy(data1_hbm.at[pl.ds(0, CHUNK)], buf, sem).wait()

    @pl.when(c_lo < c_hi)
    def _():
        issue(c_lo, buf_v, sem0)

    @pl.when(c_lo + 1 < c_hi)
    def _():
        issue(c_lo + 1, buf2_v, sem1)

    def pair_body(p, carry):
        c0 = c_lo + 2 * p
        c1 = c0 + 1
        drain(buf_v, sem0)
        carry = process_chunk(c0, buf_v, carry)

        @pl.when(c0 + 2 < c_hi)
        def _():
            issue(c0 + 2, buf_v, sem0)

        @pl.when(c1 < c_hi)
        def _():
            drain(buf2_v, sem1)

        carry = process_chunk(c1, buf2_v, carry)

        @pl.when(c1 + 2 < c_hi)
        def _():
            issue(c1 + 2, buf2_v, sem1)

        return carry

    npairs = lax.shift_right_logical(c_hi - c_lo + 1, 1)
    final = lax.fori_loop(0, npairs, pair_body, init, unroll=False)
    last_j, *last_accs = final

    @pl.when(last_j >= 0)
    def _():
        for k in range(D_VECS):
            acc_v[last_j, pl.ds(k * LANES, LANES)] = last_accs[k]

    pltpu.sync_copy(acc_v, agg_out.at[pl.ds(s_lo, SEGS_PER_WORKER)])


@functools.cache
def _sc_segment_sum():
    return pl.kernel(
        _sc_body,
        out_type=(
            jax.ShapeDtypeStruct((S, D), jnp.float32),
            jax.ShapeDtypeStruct((S,), jnp.float32),
        ),
        mesh=plsc.VectorSubcoreMesh(core_axis_name="c", subcore_axis_name="s",
                                    num_cores=NUM_CORES,
                                    num_subcores=NUM_SUBCORES),
        compiler_params=pltpu.CompilerParams(needs_layout_passes=False),
        scratch_types=[
            pltpu.VMEM((N + LANES,), jnp.int32),
            pltpu.VMEM((CHUNK, D), jnp.float32),
            pltpu.VMEM((CHUNK, D), jnp.float32),
            pltpu.VMEM((SEGS_PER_WORKER, D), jnp.float32),
            pltpu.VMEM((SEGS_PER_WORKER,), jnp.float32),
            pltpu.SemaphoreType.DMA,
            pltpu.SemaphoreType.DMA,
        ],
    )


def _tc_body(agg_ref, cnt_ref, d2_ref, out_ref):
    mat = lax.dot_general(agg_ref[...], d2_ref[...],
                          dimension_numbers=(((1,), (1,)), ((), ())),
                          preferred_element_type=jnp.float32)
    out_ref[...] = mat / jnp.maximum(cnt_ref[...], 1.0)


_tc_matmul = pl.pallas_call(
    _tc_body,
    out_shape=jax.ShapeDtypeStruct((S, M), jnp.float32),
)


@jax.jit
def kernel(data1, data2, segment_ids):
    seg = segment_ids.astype(jnp.int32)
    z_acc = jnp.zeros((SEGS_PER_WORKER, D), jnp.float32)
    agg, cnt = _sc_segment_sum()(data1, seg, z_acc)
    return _tc_matmul(agg, cnt.reshape(S, 1), data2)
